# baseline (device time: 793087 ns/iter reference)
import jax
import jax.numpy as jnp
from jax import lax
from jax.experimental import pallas as pl
from jax.experimental.pallas import tpu as pltpu

N_DEV = 32
GELU_C = 0.7978845608028654


def kernel(x, w_mat):
    m, k_per = x.shape
    _, n = w_mat.shape
    m_per = m // N_DEV

    def body(x_ref, w_ref, out_ref, w_bf, send_buf, recv_buf,
             send_sems, recv_sems, credit_sem):
        my = lax.axis_index("i")
        left = lax.rem(my + N_DEV - 1, N_DEV)
        right = lax.rem(my + 1, N_DEV)

        barrier = pltpu.get_barrier_semaphore()
        for nbr in (left, right):
            pl.semaphore_signal(barrier, inc=1, device_id=(nbr,),
                                device_id_type=pl.DeviceIdType.MESH)
        pl.semaphore_wait(barrier, 2)

        w_bf[:, :] = w_ref[:, :].astype(jnp.bfloat16)

        def partial_chunk(c):
            xa = x_ref[pl.ds(c * m_per, m_per), :].astype(jnp.bfloat16)
            return jnp.dot(xa, w_bf[:, :], preferred_element_type=jnp.float32)

        send_buf[0, :, :] = partial_chunk(left).astype(jnp.bfloat16)

        for s in range(N_DEV - 1):
            slot = s % 2
            rdma = pltpu.make_async_remote_copy(
                src_ref=send_buf.at[slot],
                dst_ref=recv_buf.at[slot],
                send_sem=send_sems.at[slot],
                recv_sem=recv_sems.at[slot],
                device_id=(right,),
                device_id_type=pl.DeviceIdType.MESH,
            )
            if s >= 2:
                pl.semaphore_wait(credit_sem, 1)
            rdma.start()
            c_recv = lax.rem(my + 2 * N_DEV - 2 - s, N_DEV)
            p = partial_chunk(c_recv)
            rdma.wait()
            if s <= N_DEV - 4:
                pl.semaphore_signal(credit_sem, inc=1, device_id=(left,),
                                    device_id_type=pl.DeviceIdType.MESH)
            acc = p + recv_buf[slot, :, :].astype(jnp.float32)
            if s < N_DEV - 2:
                send_buf[1 - slot, :, :] = acc.astype(jnp.bfloat16)
            else:
                y = acc
                out_ref[:, :] = 0.5 * y * (1.0 + jnp.tanh(
                    GELU_C * (y + 0.044715 * y * y * y)))

    return pl.pallas_call(
        body,
        out_shape=jax.ShapeDtypeStruct((m_per, n), jnp.float32),
        in_specs=[pl.BlockSpec(memory_space=pltpu.VMEM),
                  pl.BlockSpec(memory_space=pltpu.VMEM)],
        out_specs=pl.BlockSpec(memory_space=pltpu.VMEM),
        scratch_shapes=[
            pltpu.VMEM((k_per, n), jnp.bfloat16),
            pltpu.VMEM((2, m_per, n), jnp.bfloat16),
            pltpu.VMEM((2, m_per, n), jnp.bfloat16),
            pltpu.SemaphoreType.DMA((2,)),
            pltpu.SemaphoreType.DMA((2,)),
            pltpu.SemaphoreType.REGULAR,
        ],
        compiler_params=pltpu.CompilerParams(collective_id=0),
    )(x, w_mat)


# device time: 787085 ns/iter; 1.0076x vs baseline; 1.0076x over previous
import jax
import jax.numpy as jnp
from jax import lax
from jax.experimental import pallas as pl
from jax.experimental.pallas import tpu as pltpu

N_DEV = 32
GELU_C = 0.7978845608028654


def _gelu(y):
    return 0.5 * y * (1.0 + jnp.tanh(GELU_C * (y + 0.044715 * y * y * y)))


def kernel(x, w_mat):
    m, k_per = x.shape
    _, n = w_mat.shape
    m_per = m // N_DEV
    nh = n // 2

    def body(x_ref, w_ref, out_ref, w_bf,
             send_fwd, recv_fwd, send_bwd, recv_bwd,
             ssem_f, rsem_f, ssem_b, rsem_b, credit_f, credit_b):
        my = lax.axis_index("i")
        left = lax.rem(my + N_DEV - 1, N_DEV)
        right = lax.rem(my + 1, N_DEV)

        barrier = pltpu.get_barrier_semaphore()
        for nbr in (left, right):
            pl.semaphore_signal(barrier, inc=1, device_id=(nbr,),
                                device_id_type=pl.DeviceIdType.MESH)
        pl.semaphore_wait(barrier, 2)

        w_bf[:, :] = w_ref[:, :].astype(jnp.bfloat16)

        def x_chunk(c):
            return x_ref[pl.ds(c * m_per, m_per), :].astype(jnp.bfloat16)

        def p_lo(c):
            return jnp.dot(x_chunk(c), w_bf[:, :nh],
                           preferred_element_type=jnp.float32)

        def p_hi(c):
            return jnp.dot(x_chunk(c), w_bf[:, nh:],
                           preferred_element_type=jnp.float32)

        send_fwd[0, :, :] = p_lo(left).astype(jnp.bfloat16)
        send_bwd[0, :, :] = p_hi(right).astype(jnp.bfloat16)

        for s in range(N_DEV - 1):
            slot = s % 2
            rdma_f = pltpu.make_async_remote_copy(
                src_ref=send_fwd.at[slot],
                dst_ref=recv_fwd.at[slot],
                send_sem=ssem_f.at[slot],
                recv_sem=rsem_f.at[slot],
                device_id=(right,),
                device_id_type=pl.DeviceIdType.MESH,
            )
            rdma_b = pltpu.make_async_remote_copy(
                src_ref=send_bwd.at[slot],
                dst_ref=recv_bwd.at[slot],
                send_sem=ssem_b.at[slot],
                recv_sem=rsem_b.at[slot],
                device_id=(left,),
                device_id_type=pl.DeviceIdType.MESH,
            )
            if s >= 2:
                pl.semaphore_wait(credit_f, 1)
                pl.semaphore_wait(credit_b, 1)
            rdma_f.start()
            rdma_b.start()
            c_f = lax.rem(my + 2 * N_DEV - 2 - s, N_DEV)
            c_b = lax.rem(my + 2 + s, N_DEV)
            pf = p_lo(c_f)
            pb = p_hi(c_b)
            rdma_f.wait()
            rdma_b.wait()
            if s <= N_DEV - 4:
                pl.semaphore_signal(credit_f, inc=1, device_id=(left,),
                                    device_id_type=pl.DeviceIdType.MESH)
                pl.semaphore_signal(credit_b, inc=1, device_id=(right,),
                                    device_id_type=pl.DeviceIdType.MESH)
            acc_f = pf + recv_fwd[slot, :, :].astype(jnp.float32)
            acc_b = pb + recv_bwd[slot, :, :].astype(jnp.float32)
            if s < N_DEV - 2:
                send_fwd[1 - slot, :, :] = acc_f.astype(jnp.bfloat16)
                send_bwd[1 - slot, :, :] = acc_b.astype(jnp.bfloat16)
            else:
                out_ref[:, :nh] = _gelu(acc_f)
                out_ref[:, nh:] = _gelu(acc_b)

    return pl.pallas_call(
        body,
        out_shape=jax.ShapeDtypeStruct((m_per, n), jnp.float32),
        in_specs=[pl.BlockSpec(memory_space=pltpu.VMEM),
                  pl.BlockSpec(memory_space=pltpu.VMEM)],
        out_specs=pl.BlockSpec(memory_space=pltpu.VMEM),
        scratch_shapes=[
            pltpu.VMEM((k_per, n), jnp.bfloat16),
            pltpu.VMEM((2, m_per, nh), jnp.bfloat16),
            pltpu.VMEM((2, m_per, nh), jnp.bfloat16),
            pltpu.VMEM((2, m_per, nh), jnp.bfloat16),
            pltpu.VMEM((2, m_per, nh), jnp.bfloat16),
            pltpu.SemaphoreType.DMA((2,)),
            pltpu.SemaphoreType.DMA((2,)),
            pltpu.SemaphoreType.DMA((2,)),
            pltpu.SemaphoreType.DMA((2,)),
            pltpu.SemaphoreType.REGULAR,
            pltpu.SemaphoreType.REGULAR,
        ],
        compiler_params=pltpu.CompilerParams(collective_id=0),
    )(x, w_mat)


# device time: 445121 ns/iter; 1.7817x vs baseline; 1.7682x over previous
import jax
import jax.numpy as jnp
from jax import lax
from jax.experimental import pallas as pl
from jax.experimental.pallas import tpu as pltpu

N_DEV = 32
GELU_C = 0.7978845608028654

_PLANE_ORDER = {(0, 0): 0, (1, 0): 1, (1, 1): 2, (0, 1): 3,
                (0, 2): 4, (1, 2): 5, (1, 3): 6, (0, 3): 7}

_CELLS = [(0, 0), (1, 0), (2, 0), (3, 0), (3, 1), (2, 1), (1, 1), (1, 2),
          (2, 2), (3, 2), (3, 3), (2, 3), (1, 3), (0, 3), (0, 2), (0, 1)]

_RING_COORDS = []
for _i, (_y, _z) in enumerate(_CELLS):
    _xs = (0, 1) if _i % 2 == 0 else (1, 0)
    for _x in _xs:
        _RING_COORDS.append((_x, _y, _z))

for _k in range(N_DEV):
    _a, _b = _RING_COORDS[_k], _RING_COORDS[(_k + 1) % N_DEV]
    assert sum(abs(_a[_j] - _b[_j]) for _j in range(3)) == 1, (_k, _a, _b)

PERM = [z * 8 + _PLANE_ORDER[(x, y)] for (x, y, z) in _RING_COORDS]
assert sorted(PERM) == list(range(N_DEV))
INV = [0] * N_DEV
for _k, _p in enumerate(PERM):
    INV[_p] = _k


def _gelu(y):
    return 0.5 * y * (1.0 + jnp.tanh(GELU_C * (y + 0.044715 * y * y * y)))


def kernel(x, w_mat):
    m, k_per = x.shape
    _, n = w_mat.shape
    m_per = m // N_DEV
    nh = n // 2

    my = lax.axis_index("i")
    perm_arr = jnp.array(PERM, dtype=jnp.int32)
    inv_arr = jnp.array(INV, dtype=jnp.int32)
    r = inv_arr[my]
    k = jnp.arange(N_DEV, dtype=jnp.int32)
    fwd_chunks = perm_arr[(r - 1 - k) % N_DEV]
    bwd_chunks = perm_arr[(r + 1 + k) % N_DEV]
    idx = jnp.stack([fwd_chunks, bwd_chunks]).astype(jnp.int32)

    def body(x_ref, w_ref, idx_ref, out_ref, w_bf,
             send_fwd, recv_fwd, send_bwd, recv_bwd,
             ssem_f, rsem_f, ssem_b, rsem_b, credit_f, credit_b):
        left = idx_ref[0, 0]
        right = idx_ref[1, 0]

        barrier = pltpu.get_barrier_semaphore()
        for nbr in (left, right):
            pl.semaphore_signal(barrier, inc=1, device_id=(nbr,),
                                device_id_type=pl.DeviceIdType.MESH)
        pl.semaphore_wait(barrier, 2)

        w_bf[:, :] = w_ref[:, :].astype(jnp.bfloat16)

        def x_chunk(c):
            return x_ref[pl.ds(c * m_per, m_per), :].astype(jnp.bfloat16)

        def p_lo(c):
            return jnp.dot(x_chunk(c), w_bf[:, :nh],
                           preferred_element_type=jnp.float32)

        def p_hi(c):
            return jnp.dot(x_chunk(c), w_bf[:, nh:],
                           preferred_element_type=jnp.float32)

        send_fwd[0, :, :] = p_lo(idx_ref[0, 0]).astype(jnp.bfloat16)
        send_bwd[0, :, :] = p_hi(idx_ref[1, 0]).astype(jnp.bfloat16)

        for s in range(N_DEV - 1):
            slot = s % 2
            rdma_f = pltpu.make_async_remote_copy(
                src_ref=send_fwd.at[slot],
                dst_ref=recv_fwd.at[slot],
                send_sem=ssem_f.at[slot],
                recv_sem=rsem_f.at[slot],
                device_id=(right,),
                device_id_type=pl.DeviceIdType.MESH,
            )
            rdma_b = pltpu.make_async_remote_copy(
                src_ref=send_bwd.at[slot],
                dst_ref=recv_bwd.at[slot],
                send_sem=ssem_b.at[slot],
                recv_sem=rsem_b.at[slot],
                device_id=(left,),
                device_id_type=pl.DeviceIdType.MESH,
            )
            if s >= 2:
                pl.semaphore_wait(credit_f, 1)
                pl.semaphore_wait(credit_b, 1)
            rdma_f.start()
            rdma_b.start()
            pf = p_lo(idx_ref[0, s + 1])
            pb = p_hi(idx_ref[1, s + 1])
            rdma_f.wait()
            rdma_b.wait()
            if s <= N_DEV - 4:
                pl.semaphore_signal(credit_f, inc=1, device_id=(left,),
                                    device_id_type=pl.DeviceIdType.MESH)
                pl.semaphore_signal(credit_b, inc=1, device_id=(right,),
                                    device_id_type=pl.DeviceIdType.MESH)
            acc_f = pf + recv_fwd[slot, :, :].astype(jnp.float32)
            acc_b = pb + recv_bwd[slot, :, :].astype(jnp.float32)
            if s < N_DEV - 2:
                send_fwd[1 - slot, :, :] = acc_f.astype(jnp.bfloat16)
                send_bwd[1 - slot, :, :] = acc_b.astype(jnp.bfloat16)
            else:
                out_ref[:, :nh] = _gelu(acc_f)
                out_ref[:, nh:] = _gelu(acc_b)

    return pl.pallas_call(
        body,
        out_shape=jax.ShapeDtypeStruct((m_per, n), jnp.float32),
        in_specs=[pl.BlockSpec(memory_space=pltpu.VMEM),
                  pl.BlockSpec(memory_space=pltpu.VMEM),
                  pl.BlockSpec(memory_space=pltpu.SMEM)],
        out_specs=pl.BlockSpec(memory_space=pltpu.VMEM),
        scratch_shapes=[
            pltpu.VMEM((k_per, n), jnp.bfloat16),
            pltpu.VMEM((2, m_per, nh), jnp.bfloat16),
            pltpu.VMEM((2, m_per, nh), jnp.bfloat16),
            pltpu.VMEM((2, m_per, nh), jnp.bfloat16),
            pltpu.VMEM((2, m_per, nh), jnp.bfloat16),
            pltpu.SemaphoreType.DMA((2,)),
            pltpu.SemaphoreType.DMA((2,)),
            pltpu.SemaphoreType.DMA((2,)),
            pltpu.SemaphoreType.DMA((2,)),
            pltpu.SemaphoreType.REGULAR,
            pltpu.SemaphoreType.REGULAR,
        ],
        compiler_params=pltpu.CompilerParams(collective_id=0),
    )(x, w_mat, idx)
